# baseline (speedup 1.0000x reference)
import functools

import jax
import jax.numpy as jnp
from jax import lax
from jax.experimental import pallas as pl
from jax.experimental.pallas import tpu as pltpu
from jax.experimental.pallas import tpu_sc as plsc

H = 16
N = 1025
F = 3972
FPAD = 4096
NB = 1024
BR = 8
RPW = 32
EP = 1152
ER = 1032


def _sc_body(table_hbm, idx_hbm, irow_hbm, icol_hbm, out_hbm,
             table_v, idx_v0, idx_v1, out_v0, out_v1,
             edge_r, edge_c, icol_v,
             insem0, insem1, outsem0, outsem1):
    c = lax.axis_index("c")
    s = lax.axis_index("s")
    wid = s * 2 + c
    r_base = wid * RPW
    idx_b = (idx_v0, idx_v1)
    out_b = (out_v0, out_v1)
    insems = (insem0, insem1)
    outsems = (outsem0, outsem1)
    lane16 = lax.iota(jnp.int32, 16)
    zeros16 = jnp.zeros((16,), jnp.int32)

    pltpu.sync_copy(table_hbm, table_v)

    def in_copy(blk, b):
        pltpu.async_copy(
            idx_hbm.at[pl.ds(r_base + BR * blk, BR), pl.ds(0, NB)],
            idx_b[b], insems[b])

    def in_wait(b):
        pltpu.make_async_copy(idx_hbm.at[pl.ds(0, BR), pl.ds(0, NB)],
                              idx_b[b], insems[b]).wait()

    def out_wait(b, n):
        pltpu.make_async_copy(
            out_hbm.at[pl.ds(0, n), pl.ds(0, BR), pl.ds(0, 128)],
            out_b[b], outsems[b]).wait()

    in_copy(0, 0)
    in_copy(1, 1)

    def blk_pair(p, _):
        for bb in range(2):
            blk = 2 * p + bb
            rows = r_base + BR * blk
            iv = idx_b[bb]
            in_wait(bb)

            def tpair(tp, _):
                for tb in range(2):
                    t = 2 * tp + tb
                    ov = out_b[tb]

                    @pl.when((blk > 0) | (tp > 0))
                    def _():
                        out_wait(tb, H)

                    for r in range(BR):
                        @plsc.parallel_loop(0, 8, unroll=8)
                        def _(m):
                            ix = iv[r, pl.ds(t * 128 + m * 16, 16)]
                            for h in range(H):
                                ov[h, r, pl.ds(m * 16, 16)] = \
                                    plsc.load_gather(table_v, [ix + h * FPAD])

                    for h in range(H):
                        pltpu.async_copy(
                            ov.at[h],
                            out_hbm.at[h, pl.ds(rows, BR),
                                       pl.ds(t * 128, 128)],
                            outsems[tb])
                return 0

            lax.fori_loop(0, 4, tpair, 0)

            @pl.when(blk <= 1)
            def _():
                in_copy(blk + 2, bb)
        return 0

    lax.fori_loop(0, 2, blk_pair, 0)
    out_wait(0, H)
    out_wait(1, H)

    @pl.when(wid < H)
    def _():
        h = wid
        pltpu.sync_copy(irow_hbm, icol_v)
        for j in range(NB // 16):
            ix = icol_v[pl.ds(j * 16, 16)]
            edge_r[0, pl.ds(j * 16, 16)] = plsc.load_gather(
                table_v, [ix + h * FPAD])
        pltpu.sync_copy(edge_r.at[pl.ds(0, 1)],
                        out_hbm.at[h, pl.ds(NB, 1), pl.ds(0, NB)])

    @pl.when(wid >= H)
    def _():
        h = wid - H
        pltpu.sync_copy(icol_hbm, icol_v)
        for g in range(64):
            r0 = 16 * g
            gb = g % 2
            if g >= 2:
                pltpu.make_async_copy(
                    edge_c.at[gb],
                    out_hbm.at[0, pl.ds(0, 16), pl.ds(NB, 1)],
                    outsems[gb]).wait()
            ix = icol_v[pl.ds(r0, 16)]
            vals = plsc.load_gather(table_v, [ix + h * FPAD])
            plsc.store_scatter(edge_c.at[gb], [lane16, zeros16], vals)
            pltpu.async_copy(
                edge_c.at[gb],
                out_hbm.at[h, pl.ds(r0, 16), pl.ds(NB, 1)], outsems[gb])
        for gb in range(2):
            pltpu.make_async_copy(
                edge_c.at[gb],
                out_hbm.at[0, pl.ds(0, 16), pl.ds(NB, 1)],
                outsems[gb]).wait()
        ixc = icol_v[pl.ds(NB, 16)]
        valc = plsc.load_gather(table_v, [ixc + h * FPAD])
        plsc.store_scatter(edge_c.at[0], [zeros16, zeros16], valc,
                           mask=lane16 == 0)
        pltpu.sync_copy(edge_c.at[0].at[pl.ds(0, 1)],
                        out_hbm.at[h, pl.ds(NB, 1), pl.ds(NB, 1)])


@jax.jit
def _rpb(table_t, idx, irow, icol):
    mesh = plsc.VectorSubcoreMesh(
        core_axis_name="c", subcore_axis_name="s", num_cores=2,
        num_subcores=16)
    return pl.kernel(
        _sc_body,
        out_type=jax.ShapeDtypeStruct((H, N, N), jnp.float32),
        mesh=mesh,
        compiler_params=pltpu.CompilerParams(
            needs_layout_passes=False, use_tc_tiling_on_sc=True),
        scratch_types=[
            pltpu.VMEM((H * FPAD,), jnp.float32),
            pltpu.VMEM((BR, NB), jnp.int32),
            pltpu.VMEM((BR, NB), jnp.int32),
            pltpu.VMEM((H, BR, 128), jnp.float32),
            pltpu.VMEM((H, BR, 128), jnp.float32),
            pltpu.VMEM((BR, NB), jnp.float32),
            pltpu.VMEM((2, 16, 1), jnp.float32),
            pltpu.VMEM((EP,), jnp.int32),
            pltpu.SemaphoreType.DMA,
            pltpu.SemaphoreType.DMA,
            pltpu.SemaphoreType.DMA,
            pltpu.SemaphoreType.DMA,
        ],
    )(table_t, idx, irow, icol)


def kernel(relative_position_bias_table, relative_position_index):
    table_t = jnp.pad(relative_position_bias_table.T,
                      ((0, 0), (0, FPAD - F))).reshape(-1)
    irow = jnp.pad(relative_position_index[N - 1, :], (0, EP - N))
    icol = jnp.pad(relative_position_index[:, N - 1], (0, EP - N))
    return _rpb(table_t, relative_position_index, irow, icol)


# speedup vs baseline: 1.2175x; 1.2175x over previous
import functools

import jax
import jax.numpy as jnp
from jax import lax
from jax.experimental import pallas as pl
from jax.experimental.pallas import tpu as pltpu
from jax.experimental.pallas import tpu_sc as plsc

H = 16
N = 1025
F = 3972
FPAD = 4096
NB = 1024
BR = 8
RPW = 32
EP = 1152
ER = 1032


def _sc_body(table_hbm, idx_hbm, irow_hbm, icol_hbm, out_hbm,
             table_v, idx_v0, idx_v1, out_v0, out_v1,
             edge_r, edge_c, icol_v,
             insem0, insem1, outsem0, outsem1):
    c = lax.axis_index("c")
    s = lax.axis_index("s")
    wid = s * 2 + c
    r_base = wid * RPW
    idx_b = (idx_v0, idx_v1)
    out_b = (out_v0, out_v1)
    insems = (insem0, insem1)
    outsems = (outsem0, outsem1)
    lane16 = lax.iota(jnp.int32, 16)
    zeros16 = jnp.zeros((16,), jnp.int32)

    pltpu.sync_copy(table_hbm, table_v)

    def in_copy(blk, b):
        pltpu.async_copy(
            idx_hbm.at[pl.ds(r_base + BR * blk, BR), pl.ds(0, NB)],
            idx_b[b], insems[b])

    def in_wait(b):
        pltpu.make_async_copy(idx_hbm.at[pl.ds(0, BR), pl.ds(0, NB)],
                              idx_b[b], insems[b]).wait()

    def out_wait(b, n):
        pltpu.make_async_copy(
            out_hbm.at[pl.ds(0, n), pl.ds(0, BR), pl.ds(0, 128)],
            out_b[b], outsems[b]).wait()

    in_copy(0, 0)
    in_copy(1, 1)

    def blk_pair(p, _):
        for bb in range(2):
            blk = 2 * p + bb
            rows = r_base + BR * blk
            iv = idx_b[bb]
            in_wait(bb)

            def tpair(tp, _):
                for tb in range(2):
                    t = 2 * tp + tb
                    ov = out_b[tb]

                    @pl.when((blk > 0) | (tp > 0))
                    def _():
                        out_wait(tb, H)

                    for r in range(BR):
                        @plsc.parallel_loop(0, 8, unroll=4)
                        def _(m):
                            ix = iv[r, pl.ds(t * 128 + m * 16, 16)]
                            for h in range(H):
                                ov[h, r, pl.ds(m * 16, 16)] = \
                                    plsc.load_gather(table_v, [ix + h * FPAD])

                    for h in range(H):
                        pltpu.async_copy(
                            ov.at[h],
                            out_hbm.at[h, pl.ds(rows, BR),
                                       pl.ds(t * 128, 128)],
                            outsems[tb])
                return 0

            lax.fori_loop(0, 4, tpair, 0)

            @pl.when(blk <= 1)
            def _():
                in_copy(blk + 2, bb)
        return 0

    lax.fori_loop(0, 2, blk_pair, 0)
    out_wait(0, H)
    out_wait(1, H)

    @pl.when(wid < H)
    def _():
        h = wid
        pltpu.sync_copy(irow_hbm, icol_v)
        for j in range(NB // 16):
            ix = icol_v[pl.ds(j * 16, 16)]
            edge_r[0, pl.ds(j * 16, 16)] = plsc.load_gather(
                table_v, [ix + h * FPAD])
        pltpu.sync_copy(edge_r.at[pl.ds(0, 1)],
                        out_hbm.at[h, pl.ds(NB, 1), pl.ds(0, NB)])

    @pl.when(wid >= H)
    def _():
        h = wid - H
        pltpu.sync_copy(icol_hbm, icol_v)
        for g in range(64):
            r0 = 16 * g
            gb = g % 2
            if g >= 2:
                pltpu.make_async_copy(
                    edge_c.at[gb],
                    out_hbm.at[0, pl.ds(0, 16), pl.ds(NB, 1)],
                    outsems[gb]).wait()
            ix = icol_v[pl.ds(r0, 16)]
            vals = plsc.load_gather(table_v, [ix + h * FPAD])
            plsc.store_scatter(edge_c.at[gb], [lane16, zeros16], vals)
            pltpu.async_copy(
                edge_c.at[gb],
                out_hbm.at[h, pl.ds(r0, 16), pl.ds(NB, 1)], outsems[gb])
        for gb in range(2):
            pltpu.make_async_copy(
                edge_c.at[gb],
                out_hbm.at[0, pl.ds(0, 16), pl.ds(NB, 1)],
                outsems[gb]).wait()
        ixc = icol_v[pl.ds(NB, 16)]
        valc = plsc.load_gather(table_v, [ixc + h * FPAD])
        plsc.store_scatter(edge_c.at[0], [zeros16, zeros16], valc,
                           mask=lane16 == 0)
        pltpu.sync_copy(edge_c.at[0].at[pl.ds(0, 1)],
                        out_hbm.at[h, pl.ds(NB, 1), pl.ds(NB, 1)])


@jax.jit
def _rpb(table_t, idx, irow, icol):
    mesh = plsc.VectorSubcoreMesh(
        core_axis_name="c", subcore_axis_name="s", num_cores=2,
        num_subcores=16)
    return pl.kernel(
        _sc_body,
        out_type=jax.ShapeDtypeStruct((H, N, N), jnp.float32),
        mesh=mesh,
        compiler_params=pltpu.CompilerParams(
            needs_layout_passes=False, use_tc_tiling_on_sc=True),
        scratch_types=[
            pltpu.VMEM((H * FPAD,), jnp.float32),
            pltpu.VMEM((BR, NB), jnp.int32),
            pltpu.VMEM((BR, NB), jnp.int32),
            pltpu.VMEM((H, BR, 128), jnp.float32),
            pltpu.VMEM((H, BR, 128), jnp.float32),
            pltpu.VMEM((BR, NB), jnp.float32),
            pltpu.VMEM((2, 16, 1), jnp.float32),
            pltpu.VMEM((EP,), jnp.int32),
            pltpu.SemaphoreType.DMA,
            pltpu.SemaphoreType.DMA,
            pltpu.SemaphoreType.DMA,
            pltpu.SemaphoreType.DMA,
        ],
    )(table_t, idx, irow, icol)


def kernel(relative_position_bias_table, relative_position_index):
    table_t = jnp.pad(relative_position_bias_table.T,
                      ((0, 0), (0, FPAD - F))).reshape(-1)
    irow = jnp.pad(relative_position_index[N - 1, :], (0, EP - N))
    icol = jnp.pad(relative_position_index[:, N - 1], (0, EP - N))
    return _rpb(table_t, relative_position_index, irow, icol)


# speedup vs baseline: 1.3362x; 1.0975x over previous
import functools

import jax
import jax.numpy as jnp
from jax import lax
from jax.experimental import pallas as pl
from jax.experimental.pallas import tpu as pltpu
from jax.experimental.pallas import tpu_sc as plsc

H = 16
N = 1025
F = 3972
FPAD = 4096
NB = 1024
BR = 8
RPW = 32
EP = 1152
ER = 1032


def _sc_body(table_hbm, idx_hbm, irow_hbm, icol_hbm, out_hbm,
             table_v, idx_v0, idx_v1, out_v0, out_v1,
             edge_r, edge_c, icol_v,
             insem0, insem1, outsem0, outsem1):
    c = lax.axis_index("c")
    s = lax.axis_index("s")
    wid = s * 2 + c
    r_base = wid * RPW
    idx_b = (idx_v0, idx_v1)
    out_b = (out_v0, out_v1)
    insems = (insem0, insem1)
    outsems = (outsem0, outsem1)
    lane16 = lax.iota(jnp.int32, 16)
    zeros16 = jnp.zeros((16,), jnp.int32)

    pltpu.sync_copy(table_hbm, table_v)
    tabs = [table_v.at[pl.ds(h * FPAD, FPAD)] for h in range(H)]

    def in_copy(blk, b):
        pltpu.async_copy(
            idx_hbm.at[pl.ds(r_base + BR * blk, BR), pl.ds(0, NB)],
            idx_b[b], insems[b])

    def in_wait(b):
        pltpu.make_async_copy(idx_hbm.at[pl.ds(0, BR), pl.ds(0, NB)],
                              idx_b[b], insems[b]).wait()

    def out_wait(b, n):
        pltpu.make_async_copy(
            out_hbm.at[pl.ds(0, n), pl.ds(0, BR), pl.ds(0, 128)],
            out_b[b], outsems[b]).wait()

    in_copy(0, 0)
    in_copy(1, 1)

    def blk_pair(p, _):
        for bb in range(2):
            blk = 2 * p + bb
            rows = r_base + BR * blk
            iv = idx_b[bb]
            in_wait(bb)

            def tpair(tp, _):
                for tb in range(2):
                    t = 2 * tp + tb
                    ov = out_b[tb]

                    @pl.when((blk > 0) | (tp > 0))
                    def _():
                        out_wait(tb, H)

                    for r in range(BR):
                        @plsc.parallel_loop(0, 8, unroll=4)
                        def _(m):
                            ix = iv[r, pl.ds(t * 128 + m * 16, 16)]
                            for h in range(H):
                                ov[h, r, pl.ds(m * 16, 16)] = \
                                    plsc.load_gather(tabs[h], [ix])

                    for h in range(H):
                        pltpu.async_copy(
                            ov.at[h],
                            out_hbm.at[h, pl.ds(rows, BR),
                                       pl.ds(t * 128, 128)],
                            outsems[tb])
                return 0

            lax.fori_loop(0, 4, tpair, 0)

            @pl.when(blk <= 1)
            def _():
                in_copy(blk + 2, bb)
        return 0

    lax.fori_loop(0, 2, blk_pair, 0)
    out_wait(0, H)
    out_wait(1, H)

    @pl.when(wid < H)
    def _():
        h = wid
        pltpu.sync_copy(irow_hbm, icol_v)
        for j in range(NB // 16):
            ix = icol_v[pl.ds(j * 16, 16)]
            edge_r[0, pl.ds(j * 16, 16)] = plsc.load_gather(
                table_v, [ix + h * FPAD])
        pltpu.sync_copy(edge_r.at[pl.ds(0, 1)],
                        out_hbm.at[h, pl.ds(NB, 1), pl.ds(0, NB)])

    @pl.when(wid >= H)
    def _():
        h = wid - H
        pltpu.sync_copy(icol_hbm, icol_v)
        for g in range(64):
            r0 = 16 * g
            gb = g % 2
            if g >= 2:
                pltpu.make_async_copy(
                    edge_c.at[gb],
                    out_hbm.at[0, pl.ds(0, 16), pl.ds(NB, 1)],
                    outsems[gb]).wait()
            ix = icol_v[pl.ds(r0, 16)]
            vals = plsc.load_gather(table_v, [ix + h * FPAD])
            plsc.store_scatter(edge_c.at[gb], [lane16, zeros16], vals)
            pltpu.async_copy(
                edge_c.at[gb],
                out_hbm.at[h, pl.ds(r0, 16), pl.ds(NB, 1)], outsems[gb])
        for gb in range(2):
            pltpu.make_async_copy(
                edge_c.at[gb],
                out_hbm.at[0, pl.ds(0, 16), pl.ds(NB, 1)],
                outsems[gb]).wait()
        ixc = icol_v[pl.ds(NB, 16)]
        valc = plsc.load_gather(table_v, [ixc + h * FPAD])
        plsc.store_scatter(edge_c.at[0], [zeros16, zeros16], valc,
                           mask=lane16 == 0)
        pltpu.sync_copy(edge_c.at[0].at[pl.ds(0, 1)],
                        out_hbm.at[h, pl.ds(NB, 1), pl.ds(NB, 1)])


@jax.jit
def _rpb(table_t, idx, irow, icol):
    mesh = plsc.VectorSubcoreMesh(
        core_axis_name="c", subcore_axis_name="s", num_cores=2,
        num_subcores=16)
    return pl.kernel(
        _sc_body,
        out_type=jax.ShapeDtypeStruct((H, N, N), jnp.float32),
        mesh=mesh,
        compiler_params=pltpu.CompilerParams(
            needs_layout_passes=False, use_tc_tiling_on_sc=True),
        scratch_types=[
            pltpu.VMEM((H * FPAD,), jnp.float32),
            pltpu.VMEM((BR, NB), jnp.int32),
            pltpu.VMEM((BR, NB), jnp.int32),
            pltpu.VMEM((H, BR, 128), jnp.float32),
            pltpu.VMEM((H, BR, 128), jnp.float32),
            pltpu.VMEM((BR, NB), jnp.float32),
            pltpu.VMEM((2, 16, 1), jnp.float32),
            pltpu.VMEM((EP,), jnp.int32),
            pltpu.SemaphoreType.DMA,
            pltpu.SemaphoreType.DMA,
            pltpu.SemaphoreType.DMA,
            pltpu.SemaphoreType.DMA,
        ],
    )(table_t, idx, irow, icol)


def kernel(relative_position_bias_table, relative_position_index):
    table_t = jnp.pad(relative_position_bias_table.T,
                      ((0, 0), (0, FPAD - F))).reshape(-1)
    irow = jnp.pad(relative_position_index[N - 1, :], (0, EP - N))
    icol = jnp.pad(relative_position_index[:, N - 1], (0, EP - N))
    return _rpb(table_t, relative_position_index, irow, icol)


# speedup vs baseline: 1.3413x; 1.0038x over previous
import jax
import jax.numpy as jnp
from jax import lax
from jax.experimental import pallas as pl
from jax.experimental.pallas import tpu as pltpu
from jax.experimental.pallas import tpu_sc as plsc

H = 16
N = 1025
F = 3972
FPAD = 4096
NB = 1024
BR = 8
RPW = 32
EP = 1152
ER = 1032


def _sc_body(table_hbm, idx_hbm, irow_hbm, icol_hbm, out_hbm,
             table_v, idx_v0, idx_v1, out_v0, out_v1,
             edge_r, edge_c, icol_v,
             insem0, insem1, outsem0, outsem1):
    c = lax.axis_index("c")
    s = lax.axis_index("s")
    wid = s * 2 + c
    r_base = wid * RPW
    idx_b = (idx_v0, idx_v1)
    out_b = (out_v0, out_v1)
    insems = (insem0, insem1)
    outsems = (outsem0, outsem1)
    lane16 = lax.iota(jnp.int32, 16)
    zeros16 = jnp.zeros((16,), jnp.int32)

    pltpu.sync_copy(table_hbm, table_v)
    tabs = [table_v.at[pl.ds(h * FPAD, FPAD)] for h in range(H)]

    def in_copy(blk, b):
        pltpu.async_copy(
            idx_hbm.at[pl.ds(r_base + BR * blk, BR), pl.ds(0, NB)],
            idx_b[b], insems[b])

    def in_wait(b):
        pltpu.make_async_copy(idx_hbm.at[pl.ds(0, BR), pl.ds(0, NB)],
                              idx_b[b], insems[b]).wait()

    def out_wait(b, n):
        pltpu.make_async_copy(
            out_hbm.at[pl.ds(0, n), pl.ds(0, BR), pl.ds(0, 128)],
            out_b[b], outsems[b]).wait()

    in_copy(0, 0)
    in_copy(1, 1)

    def blk_pair(p, _):
        for bb in range(2):
            blk = 2 * p + bb
            rows = r_base + BR * blk
            iv = idx_b[bb]
            in_wait(bb)

            def tpair(tp, _):
                for tb in range(2):
                    t = 2 * tp + tb
                    ov = out_b[tb]

                    @pl.when((blk > 0) | (tp > 0))
                    def _():
                        out_wait(tb, H)

                    for r in range(BR):
                        @plsc.parallel_loop(0, 8, unroll=4)
                        def _(m):
                            ix = iv[r, pl.ds(t * 128 + m * 16, 16)]
                            for h in range(H):
                                ov[h, r, pl.ds(m * 16, 16)] = \
                                    plsc.load_gather(tabs[h], [ix])

                    for h in range(H):
                        pltpu.async_copy(
                            ov.at[h],
                            out_hbm.at[h, pl.ds(rows, BR),
                                       pl.ds(t * 128, 128)],
                            outsems[tb])
                return 0

            lax.fori_loop(0, 4, tpair, 0)

            @pl.when(blk <= 1)
            def _():
                in_copy(blk + 2, bb)
        return 0

    lax.fori_loop(0, 2, blk_pair, 0)
    out_wait(0, H)
    out_wait(1, H)

    @pl.when(wid < H)
    def _():
        h = wid
        pltpu.sync_copy(irow_hbm, icol_v)
        for j in range(NB // 16):
            ix = icol_v[pl.ds(j * 16, 16)]
            edge_r[0, pl.ds(j * 16, 16)] = plsc.load_gather(
                table_v, [ix + h * FPAD])
        pltpu.sync_copy(edge_r.at[pl.ds(0, 1)],
                        out_hbm.at[h, pl.ds(NB, 1), pl.ds(0, NB)])

    @pl.when(wid >= H)
    def _():
        h = wid - H
        pltpu.sync_copy(icol_hbm, icol_v)
        for g in range(64):
            r0 = 16 * g
            gb = g % 2
            if g >= 2:
                pltpu.make_async_copy(
                    edge_c.at[gb],
                    out_hbm.at[0, pl.ds(0, 16), pl.ds(NB, 1)],
                    outsems[gb]).wait()
            ix = icol_v[pl.ds(r0, 16)]
            vals = plsc.load_gather(table_v, [ix + h * FPAD])
            plsc.store_scatter(edge_c.at[gb], [lane16, zeros16], vals)
            pltpu.async_copy(
                edge_c.at[gb],
                out_hbm.at[h, pl.ds(r0, 16), pl.ds(NB, 1)], outsems[gb])
        for gb in range(2):
            pltpu.make_async_copy(
                edge_c.at[gb],
                out_hbm.at[0, pl.ds(0, 16), pl.ds(NB, 1)],
                outsems[gb]).wait()
        ixc = icol_v[pl.ds(NB, 16)]
        valc = plsc.load_gather(table_v, [ixc + h * FPAD])
        plsc.store_scatter(edge_c.at[0], [zeros16, zeros16], valc,
                           mask=lane16 == 0)
        pltpu.sync_copy(edge_c.at[0].at[pl.ds(0, 1)],
                        out_hbm.at[h, pl.ds(NB, 1), pl.ds(NB, 1)])


@jax.jit
def _rpb(table_t, idx, irow, icol):
    mesh = plsc.VectorSubcoreMesh(
        core_axis_name="c", subcore_axis_name="s", num_cores=2,
        num_subcores=16)
    return pl.kernel(
        _sc_body,
        out_type=jax.ShapeDtypeStruct((H, N, N), jnp.float32),
        mesh=mesh,
        compiler_params=pltpu.CompilerParams(
            needs_layout_passes=False, use_tc_tiling_on_sc=True),
        scratch_types=[
            pltpu.VMEM((H * FPAD,), jnp.float32),
            pltpu.VMEM((BR, NB), jnp.int32),
            pltpu.VMEM((BR, NB), jnp.int32),
            pltpu.VMEM((H, BR, 128), jnp.float32),
            pltpu.VMEM((H, BR, 128), jnp.float32),
            pltpu.VMEM((BR, NB), jnp.float32),
            pltpu.VMEM((2, 16, 1), jnp.float32),
            pltpu.VMEM((EP,), jnp.int32),
            pltpu.SemaphoreType.DMA,
            pltpu.SemaphoreType.DMA,
            pltpu.SemaphoreType.DMA,
            pltpu.SemaphoreType.DMA,
        ],
    )(table_t, idx, irow, icol)


def kernel(relative_position_bias_table, relative_position_index):
    table_t = jnp.pad(relative_position_bias_table.T,
                      ((0, 0), (0, FPAD - F))).reshape(-1)
    irow = jnp.pad(relative_position_index[N - 1, :], (0, EP - N))
    icol = jnp.pad(relative_position_index[:, N - 1], (0, EP - N))
    return _rpb(table_t, relative_position_index, irow, icol)
